# Initial kernel scaffold; baseline (speedup 1.0000x reference)
import functools

import jax
import jax.numpy as jnp
from jax import lax
from jax.experimental import pallas as pl
from jax.experimental.pallas import tpu as pltpu
from jax.experimental.pallas import tpu_sc as plsc

_B = 16384 * 200
_D = 32


def _make_emb():
    info = plsc.get_sparse_core_info()
    nc, ns = info.num_cores, info.num_subcores
    nw = nc * ns
    b_per_w = _B // nw
    chunk = 2048
    n_chunks = b_per_w // chunk

    mesh = plsc.VectorSubcoreMesh(core_axis_name="c", subcore_axis_name="s")

    @functools.partial(
        pl.kernel,
        out_type=jax.ShapeDtypeStruct((_B, _D), jnp.float32),
        mesh=mesh,
        scratch_types=[
            pltpu.VMEM((chunk,), jnp.int32),
            pltpu.VMEM((chunk, _D), jnp.float32),
            pltpu.SemaphoreType.DMA,
        ],
    )
    def emb(x_hbm, w_hbm, out_hbm, idx_v, rows_v, sem):
        wid = lax.axis_index("s") * nc + lax.axis_index("c")
        base = wid * b_per_w

        def body(i, carry):
            off = base + i * chunk
            pltpu.sync_copy(x_hbm.at[pl.ds(off, chunk)], idx_v)
            pltpu.async_copy(w_hbm.at[idx_v], rows_v, sem).wait()
            pltpu.sync_copy(rows_v, out_hbm.at[pl.ds(off, chunk)])
            return carry

        lax.fori_loop(0, n_chunks, body, 0)

    return emb


_emb = _make_emb()


def kernel(x, weight):
    out = _emb(x.reshape(-1), weight)
    return out.reshape(x.shape + (weight.shape[1],))


# speedup vs baseline: 4.9398x; 4.9398x over previous
import functools

import jax
import jax.numpy as jnp
from jax import lax
from jax.experimental import pallas as pl
from jax.experimental.pallas import tpu as pltpu
from jax.experimental.pallas import tpu_sc as plsc

_B = 16384 * 200
_D = 32


def _make_emb():
    info = plsc.get_sparse_core_info()
    nc, ns = info.num_cores, info.num_subcores
    nw = nc * ns
    b_per_w = _B // nw
    chunk = 2048
    n_chunks = b_per_w // chunk

    mesh = plsc.VectorSubcoreMesh(core_axis_name="c", subcore_axis_name="s")

    @functools.partial(
        pl.kernel,
        out_type=jax.ShapeDtypeStruct((_B, _D), jnp.float32),
        mesh=mesh,
        scratch_types=[
            pltpu.VMEM((chunk,), jnp.int32),
            pltpu.VMEM((chunk, _D), jnp.float32),
            pltpu.SemaphoreType.DMA,
        ],
        compiler_params=pltpu.CompilerParams(use_tc_tiling_on_sc=False),
    )
    def emb(x_hbm, w_hbm, out_hbm, idx_v, rows_v, sem):
        wid = lax.axis_index("s") * nc + lax.axis_index("c")
        base = wid * b_per_w

        def body(i, carry):
            off = base + i * chunk
            pltpu.sync_copy(x_hbm.at[pl.ds(off, chunk)], idx_v)
            pltpu.async_copy(w_hbm.at[idx_v], rows_v, sem).wait()
            pltpu.sync_copy(rows_v, out_hbm.at[pl.ds(off, chunk)])
            return carry

        lax.fori_loop(0, n_chunks, body, 0)

    return emb


_emb = _make_emb()


def kernel(x, weight):
    out = _emb(x.reshape(-1), weight)
    return out.reshape(x.shape + (weight.shape[1],))


# speedup vs baseline: 5.0431x; 1.0209x over previous
import functools

import jax
import jax.numpy as jnp
from jax import lax
from jax.experimental import pallas as pl
from jax.experimental.pallas import tpu as pltpu
from jax.experimental.pallas import tpu_sc as plsc

_B = 16384 * 200
_D = 32


def _make_emb():
    info = plsc.get_sparse_core_info()
    nc, ns = info.num_cores, info.num_subcores
    nw = nc * ns
    b_per_w = _B // nw
    chunk = 1600
    n_chunks = b_per_w // chunk
    n_steps = n_chunks // 2

    mesh = plsc.VectorSubcoreMesh(core_axis_name="c", subcore_axis_name="s")

    @functools.partial(
        pl.kernel,
        out_type=jax.ShapeDtypeStruct((_B, _D), jnp.float32),
        mesh=mesh,
        scratch_types=[
            pltpu.VMEM((chunk,), jnp.int32),
            pltpu.VMEM((chunk,), jnp.int32),
            pltpu.VMEM((chunk, _D), jnp.float32),
            pltpu.VMEM((chunk, _D), jnp.float32),
            pltpu.SemaphoreType.DMA,
            pltpu.SemaphoreType.DMA,
            pltpu.SemaphoreType.DMA,
            pltpu.SemaphoreType.DMA,
        ],
        compiler_params=pltpu.CompilerParams(use_tc_tiling_on_sc=False),
    )
    def emb(x_hbm, w_hbm, out_hbm, idx0, idx1, rows0, rows1,
            gsem0, gsem1, ssem0, ssem1):
        idx = (idx0, idx1)
        rows = (rows0, rows1)
        gsem = (gsem0, gsem1)
        ssem = (ssem0, ssem1)

        wid = lax.axis_index("s") * nc + lax.axis_index("c")
        base = wid * b_per_w

        for b in range(2):
            off = base + b * chunk
            pltpu.sync_copy(x_hbm.at[pl.ds(off, chunk)], idx[b])
            pltpu.async_copy(w_hbm.at[idx[b]], rows[b], gsem[b])

        def body(g, carry):
            for b in range(2):
                k = 2 * g + b
                off = base + k * chunk
                pltpu.make_async_copy(w_hbm.at[idx[b]], rows[b], gsem[b]).wait()
                pltpu.async_copy(rows[b], out_hbm.at[pl.ds(off, chunk)], ssem[b])
                off2 = jnp.minimum(off + 2 * chunk, _B - chunk)
                pltpu.sync_copy(x_hbm.at[pl.ds(off2, chunk)], idx[b])
                pltpu.make_async_copy(
                    rows[b], out_hbm.at[pl.ds(off, chunk)], ssem[b]).wait()

                @pl.when(k + 2 < n_chunks)
                def _():
                    pltpu.async_copy(w_hbm.at[idx[b]], rows[b], gsem[b])

            return carry

        lax.fori_loop(0, n_steps, body, 0)

    return emb


_emb = _make_emb()


def kernel(x, weight):
    out = _emb(x.reshape(-1), weight)
    return out.reshape(x.shape + (weight.shape[1],))


# speedup vs baseline: 5.0457x; 1.0005x over previous
import functools

import jax
import jax.numpy as jnp
from jax import lax
from jax.experimental import pallas as pl
from jax.experimental.pallas import tpu as pltpu
from jax.experimental.pallas import tpu_sc as plsc

_B = 16384 * 200
_D = 32

_NBUF = 4
_CHUNK = 800


def _make_emb():
    info = plsc.get_sparse_core_info()
    nc, ns = info.num_cores, info.num_subcores
    nw = nc * ns
    b_per_w = _B // nw
    chunk = _CHUNK
    nbuf = _NBUF
    n_chunks = b_per_w // chunk
    n_steps = n_chunks // nbuf

    mesh = plsc.VectorSubcoreMesh(core_axis_name="c", subcore_axis_name="s")

    scratch = (
        [pltpu.VMEM((chunk,), jnp.int32) for _ in range(nbuf)]
        + [pltpu.VMEM((chunk, _D), jnp.float32) for _ in range(nbuf)]
        + [pltpu.SemaphoreType.DMA for _ in range(2 * nbuf)]
    )

    @functools.partial(
        pl.kernel,
        out_type=jax.ShapeDtypeStruct((_B, _D), jnp.float32),
        mesh=mesh,
        scratch_types=scratch,
        compiler_params=pltpu.CompilerParams(use_tc_tiling_on_sc=False),
    )
    def emb(x_hbm, w_hbm, out_hbm, *bufs):
        idx = bufs[:nbuf]
        rows = bufs[nbuf:2 * nbuf]
        gsem = bufs[2 * nbuf:3 * nbuf]
        ssem = bufs[3 * nbuf:]

        wid = lax.axis_index("s") * nc + lax.axis_index("c")
        base = wid * b_per_w

        for b in range(nbuf):
            off = base + b * chunk
            pltpu.sync_copy(x_hbm.at[pl.ds(off, chunk)], idx[b])
            pltpu.async_copy(w_hbm.at[idx[b]], rows[b], gsem[b])

        def body(g, carry):
            for b in range(nbuf):
                k = nbuf * g + b
                off = base + k * chunk
                pltpu.make_async_copy(w_hbm.at[idx[b]], rows[b], gsem[b]).wait()
                pltpu.async_copy(rows[b], out_hbm.at[pl.ds(off, chunk)], ssem[b])
                off2 = jnp.minimum(off + nbuf * chunk, _B - chunk)
                pltpu.sync_copy(x_hbm.at[pl.ds(off2, chunk)], idx[b])
                pltpu.make_async_copy(
                    rows[b], out_hbm.at[pl.ds(off, chunk)], ssem[b]).wait()

                @pl.when(k + nbuf < n_chunks)
                def _():
                    pltpu.async_copy(w_hbm.at[idx[b]], rows[b], gsem[b])

            return carry

        lax.fori_loop(0, n_steps, body, 0)

    return emb


_emb = _make_emb()


def kernel(x, weight):
    out = _emb(x.reshape(-1), weight)
    return out.reshape(x.shape + (weight.shape[1],))
